# baseline (speedup 1.0000x reference)
import jax
import jax.numpy as jnp
from jax import lax
from jax.experimental import pallas as pl
from jax.experimental.pallas import tpu as pltpu
from jax.experimental.pallas import tpu_sc as plsc

MAXLEN = 200
EMBED = 64
ROWW = 2 * EMBED

_info = plsc.get_sparse_core_info()
NC, NS = _info.num_cores, _info.num_subcores
NW = NC * NS

CR = MAXLEN
NBUF = 2


def _body(x_hbm, tbl_hbm, pos_hbm, out_hbm, idx_v, rows_v, gsem, osem):
    wid = lax.axis_index("s") * NC + lax.axis_index("c")
    n_flat = x_hbm.shape[0]
    per_w = n_flat // NW
    n_chunks = per_w // CR
    base = wid * per_w

    def fetch(g, b):
        row0 = base + g * CR
        pltpu.sync_copy(x_hbm.at[pl.ds(row0, CR)], idx_v[b])
        pltpu.sync_copy(pos_hbm, rows_v[b])
        pltpu.async_copy(tbl_hbm.at[idx_v[b]], rows_v[b], gsem[b], add=True)

    fetch(0, 0)

    def out_slice(g):
        return out_hbm.at[pl.ds(base + g * CR, CR), pl.ds(0, EMBED)]

    def pair_body(t, carry):
        for b in range(NBUF):
            g = NBUF * t + b
            pltpu.make_async_copy(tbl_hbm.at[idx_v[b]], rows_v[b], gsem[b]).wait()
            pltpu.async_copy(rows_v[b], out_slice(g), osem[b])

            @pl.when(g + 1 < n_chunks)
            def _():
                @pl.when(g >= 1)
                def _():
                    pltpu.make_async_copy(
                        rows_v[1 - b], out_slice(0), osem[1 - b]
                    ).wait()
                fetch(g + 1, 1 - b)
        return carry

    lax.fori_loop(0, n_chunks // NBUF, pair_body, 0, unroll=False)
    for b in range(NBUF):
        pltpu.make_async_copy(rows_v[b], out_slice(0), osem[b]).wait()


@jax.jit
def kernel(x, token_table, pos_table):
    batch, seq_len = x.shape
    n_flat = batch * seq_len
    vocab = token_table.shape[0]
    x_flat = x.reshape(n_flat).astype(jnp.int32)
    tbl = lax.optimization_barrier(token_table.reshape(vocab // 2, ROWW))
    tbl = tbl.reshape(vocab, EMBED)

    mesh = plsc.VectorSubcoreMesh(core_axis_name="c", subcore_axis_name="s")
    run = pl.kernel(
        _body,
        out_type=jax.ShapeDtypeStruct((n_flat, ROWW), jnp.float32),
        mesh=mesh,
        scratch_types=[
            [pltpu.VMEM((CR,), jnp.int32) for _ in range(NBUF)],
            [pltpu.VMEM((CR, EMBED), jnp.float32) for _ in range(NBUF)],
            [pltpu.SemaphoreType.DMA for _ in range(NBUF)],
            [pltpu.SemaphoreType.DMA for _ in range(NBUF)],
        ],
        compiler_params=pltpu.CompilerParams(use_tc_tiling_on_sc=False),
    )
    out = run(x_flat, tbl, pos_table)
    return out[:, :EMBED].reshape(batch, seq_len, EMBED)
